# Initial kernel scaffold; baseline (speedup 1.0000x reference)
import functools

import jax
import jax.numpy as jnp
from jax import lax
from jax.experimental import pallas as pl
from jax.experimental.pallas import tpu as pltpu
from jax.experimental.pallas import tpu_sc as plsc

N_NODES = 10000
N_EDGES = 320000
D_NODE = 128
D_EDGE = 16
HID = 128

CH = 128
NCH = N_EDGES // CH
NW = 32
SROWS = N_NODES // 16

_mesh = plsc.VectorSubcoreMesh(core_axis_name="c", subcore_axis_name="s")


@functools.partial(
    pl.kernel,
    out_type=[jax.ShapeDtypeStruct((N_EDGES, HID), jnp.float32),
              jax.ShapeDtypeStruct((N_EDGES, HID), jnp.float32)],
    mesh=_mesh,
    scratch_types=[
        pltpu.VMEM((CH,), jnp.int32),
        pltpu.VMEM((CH,), jnp.int32),
        pltpu.VMEM((CH, HID), jnp.float32),
        pltpu.VMEM((CH, HID), jnp.float32),
        pltpu.SemaphoreType.DMA,
        pltpu.SemaphoreType.DMA,
    ],
)
def _sc_gather(pr, pc, rowi, coli, gr, gc, idr, idc, bufr, bufc, semr, semc):
    c = lax.axis_index("c")
    s = lax.axis_index("s")
    wid = s * 2 + c

    def body(j, carry):
        ch = wid + j * NW

        @pl.when(ch < NCH)
        def _():
            base = ch * CH
            pltpu.sync_copy(rowi.at[pl.ds(base, CH)], idr)
            pltpu.sync_copy(coli.at[pl.ds(base, CH)], idc)
            cpr = pltpu.async_copy(pr.at[idr], bufr, semr)
            cpc = pltpu.async_copy(pc.at[idc], bufc, semc)
            cpr.wait()
            pltpu.sync_copy(bufr, gr.at[pl.ds(base, CH)])
            cpc.wait()
            pltpu.sync_copy(bufc, gc.at[pl.ds(base, CH)])

        return carry

    lax.fori_loop(0, (NCH + NW - 1) // NW, body, 0)


def _sc_scatter_body(with_counts, *refs):
    if with_counts:
        (ev, coli, sums, cnts, idx, vals, ones, stripe, accs, accc) = refs
    else:
        (ev, coli, sums, idx, vals, stripe, accs) = refs
    c = lax.axis_index("c")
    s = lax.axis_index("s")
    wid = s * 2 + c

    def zbody(i, carry):
        stripe[i, :] = jnp.zeros((16,), jnp.float32)
        return carry

    lax.fori_loop(0, SROWS, zbody, 0)
    pltpu.sync_copy(stripe, accs.at[pl.ds(s * SROWS, SROWS)])
    if with_counts:
        pltpu.sync_copy(stripe, accc.at[pl.ds(s * SROWS, SROWS)])

        def obody(i, carry):
            ones[i, :] = jnp.full((16,), 1.0, jnp.float32)
            return carry

        lax.fori_loop(0, CH, obody, 0)
    plsc.subcore_barrier()

    def body(j, carry):
        ch = wid + j * NW

        @pl.when(ch < NCH)
        def _():
            base = ch * CH
            pltpu.sync_copy(coli.at[pl.ds(base, CH)], idx)
            pltpu.sync_copy(ev.at[pl.ds(base, CH)], vals)
            pltpu.sync_copy(vals, accs.at[idx], add=True)
            if with_counts:
                pltpu.sync_copy(ones, accc.at[idx], add=True)

        return carry

    lax.fori_loop(0, (NCH + NW - 1) // NW, body, 0)
    plsc.subcore_barrier()

    pltpu.sync_copy(accs.at[pl.ds(s * SROWS, SROWS)], stripe)
    pltpu.sync_copy(stripe, sums.at[pl.ds(c * N_NODES + s * SROWS, SROWS)])
    if with_counts:
        pltpu.sync_copy(accc.at[pl.ds(s * SROWS, SROWS)], stripe)
        pltpu.sync_copy(stripe, cnts.at[pl.ds(c * N_NODES + s * SROWS, SROWS)])


_sc_scatter_counts = functools.partial(
    pl.kernel,
    out_type=[jax.ShapeDtypeStruct((2 * N_NODES, D_EDGE), jnp.float32),
              jax.ShapeDtypeStruct((2 * N_NODES, D_EDGE), jnp.float32)],
    mesh=_mesh,
    scratch_types=[
        pltpu.VMEM((CH,), jnp.int32),
        pltpu.VMEM((CH, D_EDGE), jnp.float32),
        pltpu.VMEM((CH, D_EDGE), jnp.float32),
        pltpu.VMEM((SROWS, D_EDGE), jnp.float32),
        pltpu.VMEM_SHARED((N_NODES, D_EDGE), jnp.float32),
        pltpu.VMEM_SHARED((N_NODES, D_EDGE), jnp.float32),
    ],
)(functools.partial(_sc_scatter_body, True))

_sc_scatter_nocounts = functools.partial(
    pl.kernel,
    out_type=[jax.ShapeDtypeStruct((2 * N_NODES, D_EDGE), jnp.float32)],
    mesh=_mesh,
    scratch_types=[
        pltpu.VMEM((CH,), jnp.int32),
        pltpu.VMEM((CH, D_EDGE), jnp.float32),
        pltpu.VMEM((SROWS, D_EDGE), jnp.float32),
        pltpu.VMEM_SHARED((N_NODES, D_EDGE), jnp.float32),
    ],
)(functools.partial(_sc_scatter_body, False))


def _proj_body(x_ref, wa_ref, wb_ref, pr_ref, pc_ref):
    xb = x_ref[...]
    pr_ref[...] = jnp.dot(xb, wa_ref[...], preferred_element_type=jnp.float32)
    pc_ref[...] = jnp.dot(xb, wb_ref[...], preferred_element_type=jnp.float32)


def _proj(x, wa, wb):
    B = 2500
    return pl.pallas_call(
        _proj_body,
        grid=(N_NODES // B,),
        in_specs=[pl.BlockSpec((B, D_NODE), lambda i: (i, 0)),
                  pl.BlockSpec((D_NODE, HID), lambda i: (0, 0)),
                  pl.BlockSpec((D_NODE, HID), lambda i: (0, 0))],
        out_specs=[pl.BlockSpec((B, HID), lambda i: (i, 0)),
                   pl.BlockSpec((B, HID), lambda i: (i, 0))],
        out_shape=[jax.ShapeDtypeStruct((N_NODES, HID), jnp.float32),
                   jax.ShapeDtypeStruct((N_NODES, HID), jnp.float32)],
    )(x, wa, wb)


def _combine_body(gr_ref, gc_ref, ea_ref, wc_ref, b0_ref, w1_ref, b1_ref, out_ref):
    h = (gr_ref[...] + gc_ref[...]
         + jnp.dot(ea_ref[...], wc_ref[...], preferred_element_type=jnp.float32)
         + b0_ref[...])
    h = jnp.maximum(h, 0.0)
    out_ref[...] = jnp.dot(h, w1_ref[...], preferred_element_type=jnp.float32) + b1_ref[...]


def _combine(gr, gc, ea, wc, b0, w1, b1):
    B = 2000
    return pl.pallas_call(
        _combine_body,
        grid=(N_EDGES // B,),
        in_specs=[pl.BlockSpec((B, HID), lambda i: (i, 0)),
                  pl.BlockSpec((B, HID), lambda i: (i, 0)),
                  pl.BlockSpec((B, D_EDGE), lambda i: (i, 0)),
                  pl.BlockSpec((D_EDGE, HID), lambda i: (0, 0)),
                  pl.BlockSpec((1, HID), lambda i: (0, 0)),
                  pl.BlockSpec((HID, D_EDGE), lambda i: (0, 0)),
                  pl.BlockSpec((1, D_EDGE), lambda i: (0, 0))],
        out_specs=pl.BlockSpec((B, D_EDGE), lambda i: (i, 0)),
        out_shape=jax.ShapeDtypeStruct((N_EDGES, D_EDGE), jnp.float32),
    )(gr, gc, ea, wc, b0.reshape(1, HID), w1, b1.reshape(1, D_EDGE))


def _node_body(x_ref, s0_ref, s1_ref, c0_ref, c1_ref, wa_ref, wb_ref, b0_ref,
               w1_ref, b1_ref, out_ref):
    cnt = jnp.maximum(c0_ref[...] + c1_ref[...], 1.0)
    agg = (s0_ref[...] + s1_ref[...]) / cnt
    h = (jnp.dot(x_ref[...], wa_ref[...], preferred_element_type=jnp.float32)
         + jnp.dot(agg, wb_ref[...], preferred_element_type=jnp.float32)
         + b0_ref[...])
    h = jnp.maximum(h, 0.0)
    out_ref[...] = jnp.dot(h, w1_ref[...], preferred_element_type=jnp.float32) + b1_ref[...]


def _node(x, sums, cnts, w0, b0, w1, b1):
    B = 2500
    nb = N_NODES // B
    wa = w0[:D_NODE]
    wb = w0[D_NODE:]
    return pl.pallas_call(
        _node_body,
        grid=(nb,),
        in_specs=[pl.BlockSpec((B, D_NODE), lambda i: (i, 0)),
                  pl.BlockSpec((B, D_EDGE), lambda i: (i, 0)),
                  pl.BlockSpec((B, D_EDGE), lambda i, nb=nb: (i + nb, 0)),
                  pl.BlockSpec((B, D_EDGE), lambda i: (i, 0)),
                  pl.BlockSpec((B, D_EDGE), lambda i, nb=nb: (i + nb, 0)),
                  pl.BlockSpec((D_NODE, HID), lambda i: (0, 0)),
                  pl.BlockSpec((D_EDGE, HID), lambda i: (0, 0)),
                  pl.BlockSpec((1, HID), lambda i: (0, 0)),
                  pl.BlockSpec((HID, D_NODE), lambda i: (0, 0)),
                  pl.BlockSpec((1, D_NODE), lambda i: (0, 0))],
        out_specs=pl.BlockSpec((B, D_NODE), lambda i: (i, 0)),
        out_shape=jax.ShapeDtypeStruct((N_NODES, D_NODE), jnp.float32),
    )(x, sums, sums, cnts, cnts, wa, wb, b0.reshape(1, HID), w1,
      b1.reshape(1, D_NODE))


def _layer(x, ea, row, col, eW0, eb0, eW1, eb1, nW0, nb0, nW1, nb1, cnts):
    pr, pc = _proj(x, eW0[:D_NODE], eW0[D_NODE:2 * D_NODE])
    gr, gc = _sc_gather(pr, pc, row, col)
    e_new = _combine(gr, gc, ea, eW0[2 * D_NODE:], eb0, eW1, eb1)
    if cnts is None:
        sums, cnts = _sc_scatter_counts(e_new, col)
    else:
        (sums,) = _sc_scatter_nocounts(e_new, col)
    x_new = _node(x, sums, cnts, nW0, nb0, nW1, nb1)
    return x_new, e_new, cnts


def kernel(x, edge_attr, l0_eW0, l0_eb0, l0_eW1, l0_eb1, l0_nW0, l0_nb0,
           l0_nW1, l0_nb1, l1_eW0, l1_eb0, l1_eW1, l1_eb1, l1_nW0, l1_nb0,
           l1_nW1, l1_nb1, edge_index):
    row = edge_index[0]
    col = edge_index[1]
    x, e1, cnts = _layer(x, edge_attr, row, col, l0_eW0, l0_eb0, l0_eW1,
                         l0_eb1, l0_nW0, l0_nb0, l0_nW1, l0_nb1, None)
    x, _, _ = _layer(x, e1, row, col, l1_eW0, l1_eb0, l1_eW1, l1_eb1,
                     l1_nW0, l1_nb0, l1_nW1, l1_nb1, cnts)
    return x


# speedup vs baseline: 2.5608x; 2.5608x over previous
import functools

import jax
import jax.numpy as jnp
from jax import lax
from jax.experimental import pallas as pl
from jax.experimental.pallas import tpu as pltpu
from jax.experimental.pallas import tpu_sc as plsc

N_NODES = 10000
N_EDGES = 320000
D_NODE = 128
D_EDGE = 16
HID = 128

CH = 128
NCH = N_EDGES // CH
NW = 32
NP = 10240
SROWS = NP // 16

_mesh = plsc.VectorSubcoreMesh(core_axis_name="c", subcore_axis_name="s")


@functools.partial(
    pl.kernel,
    out_type=[jax.ShapeDtypeStruct((N_EDGES, HID), jnp.float32),
              jax.ShapeDtypeStruct((N_EDGES, HID), jnp.float32)],
    mesh=_mesh,
    scratch_types=[
        pltpu.VMEM((CH,), jnp.int32),
        pltpu.VMEM((CH,), jnp.int32),
        pltpu.VMEM((CH, HID), jnp.float32),
        pltpu.VMEM((CH, HID), jnp.float32),
        pltpu.SemaphoreType.DMA,
        pltpu.SemaphoreType.DMA,
    ],
)
def _sc_gather(pr, pc, rowi, coli, gr, gc, idr, idc, bufr, bufc, semr, semc):
    c = lax.axis_index("c")
    s = lax.axis_index("s")
    wid = s * 2 + c

    def body(j, carry):
        ch = wid + j * NW

        @pl.when(ch < NCH)
        def _():
            base = ch * CH
            pltpu.sync_copy(rowi.at[pl.ds(base, CH)], idr)
            pltpu.sync_copy(coli.at[pl.ds(base, CH)], idc)
            cpr = pltpu.async_copy(pr.at[idr], bufr, semr)
            cpc = pltpu.async_copy(pc.at[idc], bufc, semc)
            cpr.wait()
            pltpu.sync_copy(bufr, gr.at[pl.ds(base, CH)])
            cpc.wait()
            pltpu.sync_copy(bufc, gc.at[pl.ds(base, CH)])

        return carry

    lax.fori_loop(0, (NCH + NW - 1) // NW, body, 0)


def _sc_scatter_impl(ev, coli, sums, idx, vals, stage, accs):
    c = lax.axis_index("c")
    s = lax.axis_index("s")
    wid = s * 2 + c

    def zb(i, carry):
        for q in range(8):
            stage[i, pl.ds(q * 16, 16)] = jnp.zeros((16,), jnp.float32)
        return carry

    lax.fori_loop(0, CH, zb, 0)

    def zs(k, carry):
        pltpu.sync_copy(stage, accs.at[pl.ds(s * SROWS + k * CH, CH)])
        return carry

    lax.fori_loop(0, SROWS // CH, zs, 0)
    if ev is None:
        def ob(i, carry):
            stage[i, pl.ds(0, 16)] = jnp.full((16,), 1.0, jnp.float32)
            return carry

        lax.fori_loop(0, CH, ob, 0)
    plsc.subcore_barrier()

    def body(j, carry):
        ch = wid + j * NW

        @pl.when(ch < NCH)
        def _():
            base = ch * CH
            pltpu.sync_copy(coli.at[pl.ds(base, CH)], idx)
            if ev is not None:
                pltpu.sync_copy(ev.at[pl.ds(base * D_EDGE, CH * D_EDGE)], vals)

                def exp(i, carry2):
                    stage[i, pl.ds(0, 16)] = vals[pl.ds(i * D_EDGE, 16)]
                    return carry2

                lax.fori_loop(0, CH, exp, 0)
            pltpu.sync_copy(stage, accs.at[idx], add=True)

        return carry

    lax.fori_loop(0, (NCH + NW - 1) // NW, body, 0)
    plsc.subcore_barrier()

    def out(k, carry):
        pltpu.sync_copy(accs.at[pl.ds(s * SROWS + k * CH, CH)], stage)
        pltpu.sync_copy(stage, sums.at[pl.ds(c * NP + s * SROWS + k * CH, CH)])
        return carry

    lax.fori_loop(0, SROWS // CH, out, 0)


_SCATTER_SCRATCH = [
    pltpu.VMEM((CH,), jnp.int32),
    pltpu.VMEM((CH * D_EDGE,), jnp.float32),
    pltpu.VMEM((CH, 128), jnp.float32),
    pltpu.VMEM_SHARED((NP, 128), jnp.float32),
]

_SCATTER_OUT = [jax.ShapeDtypeStruct((2 * NP, 128), jnp.float32)]


@functools.partial(pl.kernel, out_type=_SCATTER_OUT, mesh=_mesh,
                   scratch_types=_SCATTER_SCRATCH)
def _sc_scatter(ev, coli, sums, idx, vals, stage, accs):
    _sc_scatter_impl(ev, coli, sums, idx, vals, stage, accs)


@functools.partial(pl.kernel, out_type=_SCATTER_OUT, mesh=_mesh,
                   scratch_types=_SCATTER_SCRATCH)
def _sc_counts(coli, cnts, idx, vals, stage, accs):
    _sc_scatter_impl(None, coli, cnts, idx, vals, stage, accs)


def _proj_body(x_ref, wa_ref, wb_ref, pr_ref, pc_ref):
    xb = x_ref[...]
    pr_ref[...] = jnp.dot(xb, wa_ref[...], preferred_element_type=jnp.float32)
    pc_ref[...] = jnp.dot(xb, wb_ref[...], preferred_element_type=jnp.float32)


def _proj(x, wa, wb):
    B = 2000
    return pl.pallas_call(
        _proj_body,
        grid=(N_NODES // B,),
        in_specs=[pl.BlockSpec((B, D_NODE), lambda i: (i, 0)),
                  pl.BlockSpec((D_NODE, HID), lambda i: (0, 0)),
                  pl.BlockSpec((D_NODE, HID), lambda i: (0, 0))],
        out_specs=[pl.BlockSpec((B, HID), lambda i: (i, 0)),
                   pl.BlockSpec((B, HID), lambda i: (i, 0))],
        out_shape=[jax.ShapeDtypeStruct((N_NODES, HID), jnp.float32),
                   jax.ShapeDtypeStruct((N_NODES, HID), jnp.float32)],
    )(x, wa, wb)


def _combine_body(gr_ref, gc_ref, ea_ref, wc_ref, b0_ref, w1_ref, b1_ref, out_ref):
    h = (gr_ref[...] + gc_ref[...]
         + jnp.dot(ea_ref[...], wc_ref[...], preferred_element_type=jnp.float32)
         + b0_ref[...])
    h = jnp.maximum(h, 0.0)
    out_ref[...] = jnp.dot(h, w1_ref[...], preferred_element_type=jnp.float32) + b1_ref[...]


def _combine(gr, gc, ea, wc, b0, w1, b1):
    B = 2000
    return pl.pallas_call(
        _combine_body,
        grid=(N_EDGES // B,),
        in_specs=[pl.BlockSpec((B, HID), lambda i: (i, 0)),
                  pl.BlockSpec((B, HID), lambda i: (i, 0)),
                  pl.BlockSpec((B, D_EDGE), lambda i: (i, 0)),
                  pl.BlockSpec((D_EDGE, HID), lambda i: (0, 0)),
                  pl.BlockSpec((1, HID), lambda i: (0, 0)),
                  pl.BlockSpec((HID, D_EDGE), lambda i: (0, 0)),
                  pl.BlockSpec((1, D_EDGE), lambda i: (0, 0))],
        out_specs=pl.BlockSpec((B, D_EDGE), lambda i: (i, 0)),
        out_shape=jax.ShapeDtypeStruct((N_EDGES, D_EDGE), jnp.float32),
    )(gr, gc, ea, wc, b0.reshape(1, HID), w1, b1.reshape(1, D_EDGE))


def _node_body(x_ref, s0_ref, s1_ref, c0_ref, c1_ref, wa_ref, wb_ref, b0_ref,
               w1_ref, b1_ref, out_ref):
    cnt = jnp.maximum(c0_ref[...] + c1_ref[...], 1.0)
    agg = (s0_ref[...] + s1_ref[...]) / cnt
    h = (jnp.dot(x_ref[...], wa_ref[...], preferred_element_type=jnp.float32)
         + jnp.dot(agg, wb_ref[...], preferred_element_type=jnp.float32)
         + b0_ref[...])
    h = jnp.maximum(h, 0.0)
    out_ref[...] = jnp.dot(h, w1_ref[...], preferred_element_type=jnp.float32) + b1_ref[...]


def _node(x, sums, cnts, w0, b0, w1, b1):
    B = 2000
    nb = N_NODES // B
    wa = w0[:D_NODE]
    wb = w0[D_NODE:]
    s0, s1 = sums[:N_NODES, :D_EDGE], sums[NP:NP + N_NODES, :D_EDGE]
    c0, c1 = cnts[:N_NODES, :D_EDGE], cnts[NP:NP + N_NODES, :D_EDGE]
    return pl.pallas_call(
        _node_body,
        grid=(nb,),
        in_specs=[pl.BlockSpec((B, D_NODE), lambda i: (i, 0)),
                  pl.BlockSpec((B, D_EDGE), lambda i: (i, 0)),
                  pl.BlockSpec((B, D_EDGE), lambda i: (i, 0)),
                  pl.BlockSpec((B, D_EDGE), lambda i: (i, 0)),
                  pl.BlockSpec((B, D_EDGE), lambda i: (i, 0)),
                  pl.BlockSpec((D_NODE, HID), lambda i: (0, 0)),
                  pl.BlockSpec((D_EDGE, HID), lambda i: (0, 0)),
                  pl.BlockSpec((1, HID), lambda i: (0, 0)),
                  pl.BlockSpec((HID, D_NODE), lambda i: (0, 0)),
                  pl.BlockSpec((1, D_NODE), lambda i: (0, 0))],
        out_specs=pl.BlockSpec((B, D_NODE), lambda i: (i, 0)),
        out_shape=jax.ShapeDtypeStruct((N_NODES, D_NODE), jnp.float32),
    )(x, s0, s1, c0, c1, wa, wb, b0.reshape(1, HID), w1,
      b1.reshape(1, D_NODE))


def _layer(x, ea, row, col, eW0, eb0, eW1, eb1, nW0, nb0, nW1, nb1, cnts):
    pr, pc = _proj(x, eW0[:D_NODE], eW0[D_NODE:2 * D_NODE])
    gr, gc = _sc_gather(pr, pc, row, col)
    e_new = _combine(gr, gc, ea, eW0[2 * D_NODE:], eb0, eW1, eb1)
    if cnts is None:
        (cnts,) = _sc_counts(col)
    (sums,) = _sc_scatter(e_new.reshape(-1), col)
    x_new = _node(x, sums, cnts, nW0, nb0, nW1, nb1)
    return x_new, e_new, cnts


def kernel(x, edge_attr, l0_eW0, l0_eb0, l0_eW1, l0_eb1, l0_nW0, l0_nb0,
           l0_nW1, l0_nb1, l1_eW0, l1_eb0, l1_eW1, l1_eb1, l1_nW0, l1_nb0,
           l1_nW1, l1_nb1, edge_index):
    row = edge_index[0]
    col = edge_index[1]
    x, e1, cnts = _layer(x, edge_attr, row, col, l0_eW0, l0_eb0, l0_eW1,
                         l0_eb1, l0_nW0, l0_nb0, l0_nW1, l0_nb1, None)
    x, _, _ = _layer(x, e1, row, col, l1_eW0, l1_eb0, l1_eW1, l1_eb1,
                     l1_nW0, l1_nb0, l1_nW1, l1_nb1, cnts)
    return x


# speedup vs baseline: 2.8394x; 1.1088x over previous
import functools

import jax
import jax.numpy as jnp
from jax import lax
from jax.experimental import pallas as pl
from jax.experimental.pallas import tpu as pltpu
from jax.experimental.pallas import tpu_sc as plsc

N_NODES = 10000
N_EDGES = 320000
D_NODE = 128
D_EDGE = 16
HID = 128

CH = 128
NCH = N_EDGES // CH
NW = 32
NP = 10240
SROWS = NP // 16

_mesh = plsc.VectorSubcoreMesh(core_axis_name="c", subcore_axis_name="s")


@functools.partial(
    pl.kernel,
    out_type=[jax.ShapeDtypeStruct((N_EDGES, HID), jnp.float32),
              jax.ShapeDtypeStruct((N_EDGES, HID), jnp.float32)],
    mesh=_mesh,
    scratch_types=[
        pltpu.VMEM((2, 2, CH), jnp.int32),
        pltpu.VMEM((2, CH, HID), jnp.float32),
        pltpu.VMEM((2, CH, HID), jnp.float32),
        pltpu.SemaphoreType.DMA, pltpu.SemaphoreType.DMA,
        pltpu.SemaphoreType.DMA, pltpu.SemaphoreType.DMA,
        pltpu.SemaphoreType.DMA, pltpu.SemaphoreType.DMA,
        pltpu.SemaphoreType.DMA, pltpu.SemaphoreType.DMA,
    ],
)
def _sc_gather(pr, pc, eidx, gr, gc, idx2, bufr, bufc,
               semr0, semr1, semc0, semc1, swr0, swr1, swc0, swc1):
    c = lax.axis_index("c")
    s = lax.axis_index("s")
    wid = s * 2 + c
    semr = (semr0, semr1)
    semc = (semc0, semc1)
    swr = (swr0, swr1)
    swc = (swc0, swc1)
    npair = ((NCH + NW - 1) // NW + 1) // 2

    def pair(jj, carry):
        for b in range(2):
            ch = wid + (jj * 2 + b) * NW

            @pl.when(ch < NCH)
            def _(b=b, ch=ch):
                base = ch * CH
                pltpu.sync_copy(eidx.at[:, pl.ds(base, CH)], idx2.at[b])
                pltpu.async_copy(pr.at[idx2.at[b, 0]], bufr.at[b], semr[b])
                pltpu.async_copy(pc.at[idx2.at[b, 1]], bufc.at[b], semc[b])

        for b in range(2):
            ch = wid + (jj * 2 + b) * NW

            @pl.when(ch < NCH)
            def _(b=b, ch=ch):
                base = ch * CH
                pltpu.make_async_copy(pr.at[idx2.at[b, 0]], bufr.at[b],
                                      semr[b]).wait()
                pltpu.async_copy(bufr.at[b], gr.at[pl.ds(base, CH)], swr[b])
                pltpu.make_async_copy(pc.at[idx2.at[b, 1]], bufc.at[b],
                                      semc[b]).wait()
                pltpu.async_copy(bufc.at[b], gc.at[pl.ds(base, CH)], swc[b])

        for b in range(2):
            ch = wid + (jj * 2 + b) * NW

            @pl.when(ch < NCH)
            def _(b=b, ch=ch):
                base = ch * CH
                pltpu.make_async_copy(bufr.at[b], gr.at[pl.ds(base, CH)],
                                      swr[b]).wait()
                pltpu.make_async_copy(bufc.at[b], gc.at[pl.ds(base, CH)],
                                      swc[b]).wait()

        return carry

    lax.fori_loop(0, npair, pair, 0)


def _sc_scatter_impl(ev, coli, sums, idx, vals, stage, accs):
    c = lax.axis_index("c")
    s = lax.axis_index("s")
    wid = s * 2 + c

    def zb(i, carry):
        for q in range(8):
            stage[i, pl.ds(q * 16, 16)] = jnp.zeros((16,), jnp.float32)
        return carry

    lax.fori_loop(0, CH, zb, 0)

    def zs(k, carry):
        pltpu.sync_copy(stage, accs.at[pl.ds(s * SROWS + k * CH, CH)])
        return carry

    lax.fori_loop(0, SROWS // CH, zs, 0)

    def ob(i, carry):
        stage[i, pl.ds(16, 16)] = jnp.full((16,), 1.0, jnp.float32)
        return carry

    lax.fori_loop(0, CH, ob, 0)
    plsc.subcore_barrier()

    def body(j, carry):
        ch = wid + j * NW

        @pl.when(ch < NCH)
        def _():
            base = ch * CH
            pltpu.sync_copy(coli.at[pl.ds(base, CH)], idx)
            pltpu.sync_copy(ev.at[pl.ds(base * D_EDGE, CH * D_EDGE)], vals)

            def exp(i, carry2):
                stage[i, pl.ds(0, 16)] = vals[pl.ds(i * D_EDGE, 16)]
                return carry2

            lax.fori_loop(0, CH, exp, 0)
            pltpu.sync_copy(stage, accs.at[idx], add=True)

        return carry

    lax.fori_loop(0, (NCH + NW - 1) // NW, body, 0)
    plsc.subcore_barrier()

    def out(k, carry):
        pltpu.sync_copy(accs.at[pl.ds(s * SROWS + k * CH, CH)], stage)
        pltpu.sync_copy(stage, sums.at[pl.ds(c * NP + s * SROWS + k * CH, CH)])
        return carry

    lax.fori_loop(0, SROWS // CH, out, 0)


_SCATTER_SCRATCH = [
    pltpu.VMEM((CH,), jnp.int32),
    pltpu.VMEM((CH * D_EDGE,), jnp.float32),
    pltpu.VMEM((CH, 128), jnp.float32),
    pltpu.VMEM_SHARED((NP, 128), jnp.float32),
]

_SCATTER_OUT = [jax.ShapeDtypeStruct((2 * NP, 128), jnp.float32)]


@functools.partial(pl.kernel, out_type=_SCATTER_OUT, mesh=_mesh,
                   scratch_types=_SCATTER_SCRATCH)
def _sc_scatter(ev, coli, sums, idx, vals, stage, accs):
    _sc_scatter_impl(ev, coli, sums, idx, vals, stage, accs)


def _proj_body(x_ref, wa_ref, wb_ref, pr_ref, pc_ref):
    xb = x_ref[...]
    pr_ref[...] = jnp.dot(xb, wa_ref[...], preferred_element_type=jnp.float32)
    pc_ref[...] = jnp.dot(xb, wb_ref[...], preferred_element_type=jnp.float32)


def _proj(x, wa, wb):
    B = 2000
    return pl.pallas_call(
        _proj_body,
        grid=(N_NODES // B,),
        in_specs=[pl.BlockSpec((B, D_NODE), lambda i: (i, 0)),
                  pl.BlockSpec((D_NODE, HID), lambda i: (0, 0)),
                  pl.BlockSpec((D_NODE, HID), lambda i: (0, 0))],
        out_specs=[pl.BlockSpec((B, HID), lambda i: (i, 0)),
                   pl.BlockSpec((B, HID), lambda i: (i, 0))],
        out_shape=[jax.ShapeDtypeStruct((N_NODES, HID), jnp.float32),
                   jax.ShapeDtypeStruct((N_NODES, HID), jnp.float32)],
    )(x, wa, wb)


def _combine_body(gr_ref, gc_ref, ea_ref, wc_ref, b0_ref, w1_ref, b1_ref, out_ref):
    h = (gr_ref[...] + gc_ref[...]
         + jnp.dot(ea_ref[...], wc_ref[...], preferred_element_type=jnp.float32)
         + b0_ref[...])
    h = jnp.maximum(h, 0.0)
    out_ref[...] = jnp.dot(h, w1_ref[...], preferred_element_type=jnp.float32) + b1_ref[...]


def _combine(gr, gc, ea, wc, b0, w1, b1):
    B = 2000
    return pl.pallas_call(
        _combine_body,
        grid=(N_EDGES // B,),
        in_specs=[pl.BlockSpec((B, HID), lambda i: (i, 0)),
                  pl.BlockSpec((B, HID), lambda i: (i, 0)),
                  pl.BlockSpec((B, D_EDGE), lambda i: (i, 0)),
                  pl.BlockSpec((D_EDGE, HID), lambda i: (0, 0)),
                  pl.BlockSpec((1, HID), lambda i: (0, 0)),
                  pl.BlockSpec((HID, D_EDGE), lambda i: (0, 0)),
                  pl.BlockSpec((1, D_EDGE), lambda i: (0, 0))],
        out_specs=pl.BlockSpec((B, D_EDGE), lambda i: (i, 0)),
        out_shape=jax.ShapeDtypeStruct((N_EDGES, D_EDGE), jnp.float32),
    )(gr, gc, ea, wc, b0.reshape(1, HID), w1, b1.reshape(1, D_EDGE))


def _node_body(x_ref, s0_ref, s1_ref, c0_ref, c1_ref, wa_ref, wb_ref, b0_ref,
               w1_ref, b1_ref, out_ref):
    cnt = jnp.maximum(c0_ref[...] + c1_ref[...], 1.0)
    agg = (s0_ref[...] + s1_ref[...]) / cnt
    h = (jnp.dot(x_ref[...], wa_ref[...], preferred_element_type=jnp.float32)
         + jnp.dot(agg, wb_ref[...], preferred_element_type=jnp.float32)
         + b0_ref[...])
    h = jnp.maximum(h, 0.0)
    out_ref[...] = jnp.dot(h, w1_ref[...], preferred_element_type=jnp.float32) + b1_ref[...]


def _node_proj_body(x_ref, s0_ref, s1_ref, c0_ref, c1_ref, wa_ref, wb_ref,
                    b0_ref, w1_ref, b1_ref, wea_ref, web_ref,
                    out_ref, pr_ref, pc_ref):
    cnt = jnp.maximum(c0_ref[...] + c1_ref[...], 1.0)
    agg = (s0_ref[...] + s1_ref[...]) / cnt
    h = (jnp.dot(x_ref[...], wa_ref[...], preferred_element_type=jnp.float32)
         + jnp.dot(agg, wb_ref[...], preferred_element_type=jnp.float32)
         + b0_ref[...])
    h = jnp.maximum(h, 0.0)
    xn = jnp.dot(h, w1_ref[...], preferred_element_type=jnp.float32) + b1_ref[...]
    out_ref[...] = xn
    pr_ref[...] = jnp.dot(xn, wea_ref[...], preferred_element_type=jnp.float32)
    pc_ref[...] = jnp.dot(xn, web_ref[...], preferred_element_type=jnp.float32)


def _node_slices(sums):
    s0 = sums[:N_NODES, :D_EDGE]
    s1 = sums[NP:NP + N_NODES, :D_EDGE]
    c0 = sums[:N_NODES, D_EDGE:2 * D_EDGE]
    c1 = sums[NP:NP + N_NODES, D_EDGE:2 * D_EDGE]
    return s0, s1, c0, c1


def _node(x, sums, w0, b0, w1, b1):
    B = 2000
    nb = N_NODES // B
    s0, s1, c0, c1 = _node_slices(sums)
    return pl.pallas_call(
        _node_body,
        grid=(nb,),
        in_specs=[pl.BlockSpec((B, D_NODE), lambda i: (i, 0)),
                  pl.BlockSpec((B, D_EDGE), lambda i: (i, 0)),
                  pl.BlockSpec((B, D_EDGE), lambda i: (i, 0)),
                  pl.BlockSpec((B, D_EDGE), lambda i: (i, 0)),
                  pl.BlockSpec((B, D_EDGE), lambda i: (i, 0)),
                  pl.BlockSpec((D_NODE, HID), lambda i: (0, 0)),
                  pl.BlockSpec((D_EDGE, HID), lambda i: (0, 0)),
                  pl.BlockSpec((1, HID), lambda i: (0, 0)),
                  pl.BlockSpec((HID, D_NODE), lambda i: (0, 0)),
                  pl.BlockSpec((1, D_NODE), lambda i: (0, 0))],
        out_specs=pl.BlockSpec((B, D_NODE), lambda i: (i, 0)),
        out_shape=jax.ShapeDtypeStruct((N_NODES, D_NODE), jnp.float32),
    )(x, s0, s1, c0, c1, w0[:D_NODE], w0[D_NODE:], b0.reshape(1, HID), w1,
      b1.reshape(1, D_NODE))


def _node_proj(x, sums, w0, b0, w1, b1, wea, web):
    B = 2000
    nb = N_NODES // B
    s0, s1, c0, c1 = _node_slices(sums)
    return pl.pallas_call(
        _node_proj_body,
        grid=(nb,),
        in_specs=[pl.BlockSpec((B, D_NODE), lambda i: (i, 0)),
                  pl.BlockSpec((B, D_EDGE), lambda i: (i, 0)),
                  pl.BlockSpec((B, D_EDGE), lambda i: (i, 0)),
                  pl.BlockSpec((B, D_EDGE), lambda i: (i, 0)),
                  pl.BlockSpec((B, D_EDGE), lambda i: (i, 0)),
                  pl.BlockSpec((D_NODE, HID), lambda i: (0, 0)),
                  pl.BlockSpec((D_EDGE, HID), lambda i: (0, 0)),
                  pl.BlockSpec((1, HID), lambda i: (0, 0)),
                  pl.BlockSpec((HID, D_NODE), lambda i: (0, 0)),
                  pl.BlockSpec((1, D_NODE), lambda i: (0, 0)),
                  pl.BlockSpec((D_NODE, HID), lambda i: (0, 0)),
                  pl.BlockSpec((D_NODE, HID), lambda i: (0, 0))],
        out_specs=[pl.BlockSpec((B, D_NODE), lambda i: (i, 0)),
                   pl.BlockSpec((B, HID), lambda i: (i, 0)),
                   pl.BlockSpec((B, HID), lambda i: (i, 0))],
        out_shape=[jax.ShapeDtypeStruct((N_NODES, D_NODE), jnp.float32),
                   jax.ShapeDtypeStruct((N_NODES, HID), jnp.float32),
                   jax.ShapeDtypeStruct((N_NODES, HID), jnp.float32)],
    )(x, s0, s1, c0, c1, w0[:D_NODE], w0[D_NODE:], b0.reshape(1, HID), w1,
      b1.reshape(1, D_NODE), wea, web)


def kernel(x, edge_attr, l0_eW0, l0_eb0, l0_eW1, l0_eb1, l0_nW0, l0_nb0,
           l0_nW1, l0_nb1, l1_eW0, l1_eb0, l1_eW1, l1_eb1, l1_nW0, l1_nb0,
           l1_nW1, l1_nb1, edge_index):
    col = edge_index[1]
    pr, pc = _proj(x, l0_eW0[:D_NODE], l0_eW0[D_NODE:2 * D_NODE])
    gr, gc = _sc_gather(pr, pc, edge_index)
    e1 = _combine(gr, gc, edge_attr, l0_eW0[2 * D_NODE:], l0_eb0, l0_eW1,
                  l0_eb1)
    (sums0,) = _sc_scatter(e1.reshape(-1), col)
    x1, pr1, pc1 = _node_proj(x, sums0, l0_nW0, l0_nb0, l0_nW1, l0_nb1,
                              l1_eW0[:D_NODE], l1_eW0[D_NODE:2 * D_NODE])
    gr1, gc1 = _sc_gather(pr1, pc1, edge_index)
    e2 = _combine(gr1, gc1, e1, l1_eW0[2 * D_NODE:], l1_eb0, l1_eW1, l1_eb1)
    (sums1,) = _sc_scatter(e2.reshape(-1), col)
    return _node(x1, sums1, l1_nW0, l1_nb0, l1_nW1, l1_nb1)


# speedup vs baseline: 2.9659x; 1.0446x over previous
import functools

import jax
import jax.numpy as jnp
from jax import lax
from jax.experimental import pallas as pl
from jax.experimental.pallas import tpu as pltpu
from jax.experimental.pallas import tpu_sc as plsc

N_NODES = 10000
N_EDGES = 320000
D_NODE = 128
D_EDGE = 16
HID = 128

CH = 128
NCH = N_EDGES // CH
NW = 32
NP = 10240
SROWS = NP // 16

_mesh = plsc.VectorSubcoreMesh(core_axis_name="c", subcore_axis_name="s")


@functools.partial(
    pl.kernel,
    out_type=[jax.ShapeDtypeStruct((N_EDGES, HID), jnp.float32),
              jax.ShapeDtypeStruct((N_EDGES, HID), jnp.float32)],
    mesh=_mesh,
    scratch_types=[
        pltpu.VMEM((2, 2, CH), jnp.int32),
        pltpu.VMEM((2, CH, HID), jnp.float32),
        pltpu.VMEM((2, CH, HID), jnp.float32),
        pltpu.SemaphoreType.DMA, pltpu.SemaphoreType.DMA,
        pltpu.SemaphoreType.DMA, pltpu.SemaphoreType.DMA,
        pltpu.SemaphoreType.DMA, pltpu.SemaphoreType.DMA,
        pltpu.SemaphoreType.DMA, pltpu.SemaphoreType.DMA,
    ],
)
def _sc_gather(pr, pc, eidx, gr, gc, idx2, bufr, bufc,
               semr0, semr1, semc0, semc1, swr0, swr1, swc0, swc1):
    c = lax.axis_index("c")
    s = lax.axis_index("s")
    wid = s * 2 + c
    semr = (semr0, semr1)
    semc = (semc0, semc1)
    swr = (swr0, swr1)
    swc = (swc0, swc1)
    npair = ((NCH + NW - 1) // NW + 1) // 2

    def pair(jj, carry):
        for b in range(2):
            ch = wid + (jj * 2 + b) * NW

            @pl.when(ch < NCH)
            def _(b=b, ch=ch):
                base = ch * CH
                pltpu.sync_copy(eidx.at[:, pl.ds(base, CH)], idx2.at[b])
                pltpu.async_copy(pr.at[idx2.at[b, 0]], bufr.at[b], semr[b])
                pltpu.async_copy(pc.at[idx2.at[b, 1]], bufc.at[b], semc[b])

        for b in range(2):
            ch = wid + (jj * 2 + b) * NW

            @pl.when(ch < NCH)
            def _(b=b, ch=ch):
                base = ch * CH
                pltpu.make_async_copy(pr.at[idx2.at[b, 0]], bufr.at[b],
                                      semr[b]).wait()
                pltpu.async_copy(bufr.at[b], gr.at[pl.ds(base, CH)], swr[b])
                pltpu.make_async_copy(pc.at[idx2.at[b, 1]], bufc.at[b],
                                      semc[b]).wait()
                pltpu.async_copy(bufc.at[b], gc.at[pl.ds(base, CH)], swc[b])

        for b in range(2):
            ch = wid + (jj * 2 + b) * NW

            @pl.when(ch < NCH)
            def _(b=b, ch=ch):
                base = ch * CH
                pltpu.make_async_copy(bufr.at[b], gr.at[pl.ds(base, CH)],
                                      swr[b]).wait()
                pltpu.make_async_copy(bufc.at[b], gc.at[pl.ds(base, CH)],
                                      swc[b]).wait()

        return carry

    lax.fori_loop(0, npair, pair, 0)


_SCATTER_SCRATCH = [
    pltpu.VMEM((2, CH), jnp.int32),
    pltpu.VMEM((2, CH * D_EDGE), jnp.float32),
    pltpu.VMEM((2, CH, 128), jnp.float32),
    pltpu.VMEM_SHARED((NP, 128), jnp.float32),
    pltpu.SemaphoreType.DMA,
    pltpu.SemaphoreType.DMA,
]

_SCATTER_OUT = [jax.ShapeDtypeStruct((2 * NP, 128), jnp.float32)]


@functools.partial(pl.kernel, out_type=_SCATTER_OUT, mesh=_mesh,
                   scratch_types=_SCATTER_SCRATCH)
def _sc_scatter(ev, coli, sums, idx, vals, stage, accs, sa0, sa1):
    c = lax.axis_index("c")
    s = lax.axis_index("s")
    wid = s * 2 + c
    sads = (sa0, sa1)

    def zb(i, carry):
        for b in range(2):
            for q in range(8):
                stage[b, i, pl.ds(q * 16, 16)] = jnp.zeros((16,), jnp.float32)
        return carry

    lax.fori_loop(0, CH, zb, 0)

    def zs(k, carry):
        pltpu.sync_copy(stage.at[0], accs.at[pl.ds(s * SROWS + k * CH, CH)])
        return carry

    lax.fori_loop(0, SROWS // CH, zs, 0)

    def ob(i, carry):
        for b in range(2):
            stage[b, i, pl.ds(16, 16)] = jnp.full((16,), 1.0, jnp.float32)
        return carry

    lax.fori_loop(0, CH, ob, 0)
    plsc.subcore_barrier()

    npair = ((NCH + NW - 1) // NW + 1) // 2

    def pair(jj, carry):
        for b in range(2):
            ch = wid + (jj * 2 + b) * NW

            @pl.when(ch < NCH)
            def _(b=b, ch=ch):
                base = ch * CH
                pltpu.sync_copy(coli.at[pl.ds(base, CH)], idx.at[b])
                pltpu.sync_copy(ev.at[pl.ds(base * D_EDGE, CH * D_EDGE)],
                                vals.at[b])

                def exp(i, carry2):
                    stage[b, i, pl.ds(0, 16)] = vals[b, pl.ds(i * D_EDGE, 16)]
                    return carry2

                lax.fori_loop(0, CH, exp, 0)
                pltpu.async_copy(stage.at[b], accs.at[idx.at[b]], sads[b],
                                 add=True)

        for b in range(2):
            ch = wid + (jj * 2 + b) * NW

            @pl.when(ch < NCH)
            def _(b=b, ch=ch):
                pltpu.make_async_copy(stage.at[b], accs.at[idx.at[b]],
                                      sads[b]).wait()

        return carry

    lax.fori_loop(0, npair, pair, 0)
    plsc.subcore_barrier()

    def out(k, carry):
        pltpu.sync_copy(accs.at[pl.ds(s * SROWS + k * CH, CH)], stage.at[0])
        pltpu.sync_copy(stage.at[0],
                        sums.at[pl.ds(c * NP + s * SROWS + k * CH, CH)])
        return carry

    lax.fori_loop(0, SROWS // CH, out, 0)


def _proj_body(x_ref, wa_ref, wb_ref, pr_ref, pc_ref):
    xb = x_ref[...]
    pr_ref[...] = jnp.dot(xb, wa_ref[...], preferred_element_type=jnp.float32)
    pc_ref[...] = jnp.dot(xb, wb_ref[...], preferred_element_type=jnp.float32)


def _proj(x, wa, wb):
    B = 2000
    return pl.pallas_call(
        _proj_body,
        grid=(N_NODES // B,),
        in_specs=[pl.BlockSpec((B, D_NODE), lambda i: (i, 0)),
                  pl.BlockSpec((D_NODE, HID), lambda i: (0, 0)),
                  pl.BlockSpec((D_NODE, HID), lambda i: (0, 0))],
        out_specs=[pl.BlockSpec((B, HID), lambda i: (i, 0)),
                   pl.BlockSpec((B, HID), lambda i: (i, 0))],
        out_shape=[jax.ShapeDtypeStruct((N_NODES, HID), jnp.float32),
                   jax.ShapeDtypeStruct((N_NODES, HID), jnp.float32)],
    )(x, wa, wb)


def _combine_body(gr_ref, gc_ref, ea_ref, wc_ref, b0_ref, w1_ref, b1_ref, out_ref):
    h = (gr_ref[...] + gc_ref[...]
         + jnp.dot(ea_ref[...], wc_ref[...], preferred_element_type=jnp.float32)
         + b0_ref[...])
    h = jnp.maximum(h, 0.0)
    out_ref[...] = jnp.dot(h, w1_ref[...], preferred_element_type=jnp.float32) + b1_ref[...]


def _combine(gr, gc, ea, wc, b0, w1, b1):
    B = 2000
    return pl.pallas_call(
        _combine_body,
        grid=(N_EDGES // B,),
        in_specs=[pl.BlockSpec((B, HID), lambda i: (i, 0)),
                  pl.BlockSpec((B, HID), lambda i: (i, 0)),
                  pl.BlockSpec((B, D_EDGE), lambda i: (i, 0)),
                  pl.BlockSpec((D_EDGE, HID), lambda i: (0, 0)),
                  pl.BlockSpec((1, HID), lambda i: (0, 0)),
                  pl.BlockSpec((HID, D_EDGE), lambda i: (0, 0)),
                  pl.BlockSpec((1, D_EDGE), lambda i: (0, 0))],
        out_specs=pl.BlockSpec((B, D_EDGE), lambda i: (i, 0)),
        out_shape=jax.ShapeDtypeStruct((N_EDGES, D_EDGE), jnp.float32),
    )(gr, gc, ea, wc, b0.reshape(1, HID), w1, b1.reshape(1, D_EDGE))


def _node_body(x_ref, s0_ref, s1_ref, c0_ref, c1_ref, wa_ref, wb_ref, b0_ref,
               w1_ref, b1_ref, out_ref):
    cnt = jnp.maximum(c0_ref[...] + c1_ref[...], 1.0)
    agg = (s0_ref[...] + s1_ref[...]) / cnt
    h = (jnp.dot(x_ref[...], wa_ref[...], preferred_element_type=jnp.float32)
         + jnp.dot(agg, wb_ref[...], preferred_element_type=jnp.float32)
         + b0_ref[...])
    h = jnp.maximum(h, 0.0)
    out_ref[...] = jnp.dot(h, w1_ref[...], preferred_element_type=jnp.float32) + b1_ref[...]


def _node_proj_body(x_ref, s0_ref, s1_ref, c0_ref, c1_ref, wa_ref, wb_ref,
                    b0_ref, w1_ref, b1_ref, wea_ref, web_ref,
                    out_ref, pr_ref, pc_ref):
    cnt = jnp.maximum(c0_ref[...] + c1_ref[...], 1.0)
    agg = (s0_ref[...] + s1_ref[...]) / cnt
    h = (jnp.dot(x_ref[...], wa_ref[...], preferred_element_type=jnp.float32)
         + jnp.dot(agg, wb_ref[...], preferred_element_type=jnp.float32)
         + b0_ref[...])
    h = jnp.maximum(h, 0.0)
    xn = jnp.dot(h, w1_ref[...], preferred_element_type=jnp.float32) + b1_ref[...]
    out_ref[...] = xn
    pr_ref[...] = jnp.dot(xn, wea_ref[...], preferred_element_type=jnp.float32)
    pc_ref[...] = jnp.dot(xn, web_ref[...], preferred_element_type=jnp.float32)


def _node_slices(sums):
    s0 = sums[:N_NODES, :D_EDGE]
    s1 = sums[NP:NP + N_NODES, :D_EDGE]
    c0 = sums[:N_NODES, D_EDGE:2 * D_EDGE]
    c1 = sums[NP:NP + N_NODES, D_EDGE:2 * D_EDGE]
    return s0, s1, c0, c1


def _node(x, sums, w0, b0, w1, b1):
    B = 2000
    nb = N_NODES // B
    s0, s1, c0, c1 = _node_slices(sums)
    return pl.pallas_call(
        _node_body,
        grid=(nb,),
        in_specs=[pl.BlockSpec((B, D_NODE), lambda i: (i, 0)),
                  pl.BlockSpec((B, D_EDGE), lambda i: (i, 0)),
                  pl.BlockSpec((B, D_EDGE), lambda i: (i, 0)),
                  pl.BlockSpec((B, D_EDGE), lambda i: (i, 0)),
                  pl.BlockSpec((B, D_EDGE), lambda i: (i, 0)),
                  pl.BlockSpec((D_NODE, HID), lambda i: (0, 0)),
                  pl.BlockSpec((D_EDGE, HID), lambda i: (0, 0)),
                  pl.BlockSpec((1, HID), lambda i: (0, 0)),
                  pl.BlockSpec((HID, D_NODE), lambda i: (0, 0)),
                  pl.BlockSpec((1, D_NODE), lambda i: (0, 0))],
        out_specs=pl.BlockSpec((B, D_NODE), lambda i: (i, 0)),
        out_shape=jax.ShapeDtypeStruct((N_NODES, D_NODE), jnp.float32),
    )(x, s0, s1, c0, c1, w0[:D_NODE], w0[D_NODE:], b0.reshape(1, HID), w1,
      b1.reshape(1, D_NODE))


def _node_proj(x, sums, w0, b0, w1, b1, wea, web):
    B = 2000
    nb = N_NODES // B
    s0, s1, c0, c1 = _node_slices(sums)
    return pl.pallas_call(
        _node_proj_body,
        grid=(nb,),
        in_specs=[pl.BlockSpec((B, D_NODE), lambda i: (i, 0)),
                  pl.BlockSpec((B, D_EDGE), lambda i: (i, 0)),
                  pl.BlockSpec((B, D_EDGE), lambda i: (i, 0)),
                  pl.BlockSpec((B, D_EDGE), lambda i: (i, 0)),
                  pl.BlockSpec((B, D_EDGE), lambda i: (i, 0)),
                  pl.BlockSpec((D_NODE, HID), lambda i: (0, 0)),
                  pl.BlockSpec((D_EDGE, HID), lambda i: (0, 0)),
                  pl.BlockSpec((1, HID), lambda i: (0, 0)),
                  pl.BlockSpec((HID, D_NODE), lambda i: (0, 0)),
                  pl.BlockSpec((1, D_NODE), lambda i: (0, 0)),
                  pl.BlockSpec((D_NODE, HID), lambda i: (0, 0)),
                  pl.BlockSpec((D_NODE, HID), lambda i: (0, 0))],
        out_specs=[pl.BlockSpec((B, D_NODE), lambda i: (i, 0)),
                   pl.BlockSpec((B, HID), lambda i: (i, 0)),
                   pl.BlockSpec((B, HID), lambda i: (i, 0))],
        out_shape=[jax.ShapeDtypeStruct((N_NODES, D_NODE), jnp.float32),
                   jax.ShapeDtypeStruct((N_NODES, HID), jnp.float32),
                   jax.ShapeDtypeStruct((N_NODES, HID), jnp.float32)],
    )(x, s0, s1, c0, c1, w0[:D_NODE], w0[D_NODE:], b0.reshape(1, HID), w1,
      b1.reshape(1, D_NODE), wea, web)


def kernel(x, edge_attr, l0_eW0, l0_eb0, l0_eW1, l0_eb1, l0_nW0, l0_nb0,
           l0_nW1, l0_nb1, l1_eW0, l1_eb0, l1_eW1, l1_eb1, l1_nW0, l1_nb0,
           l1_nW1, l1_nb1, edge_index):
    col = edge_index[1]
    pr, pc = _proj(x, l0_eW0[:D_NODE], l0_eW0[D_NODE:2 * D_NODE])
    gr, gc = _sc_gather(pr, pc, edge_index)
    e1 = _combine(gr, gc, edge_attr, l0_eW0[2 * D_NODE:], l0_eb0, l0_eW1,
                  l0_eb1)
    (sums0,) = _sc_scatter(e1.reshape(-1), col)
    x1, pr1, pc1 = _node_proj(x, sums0, l0_nW0, l0_nb0, l0_nW1, l0_nb1,
                              l1_eW0[:D_NODE], l1_eW0[D_NODE:2 * D_NODE])
    gr1, gc1 = _sc_gather(pr1, pc1, edge_index)
    e2 = _combine(gr1, gc1, e1, l1_eW0[2 * D_NODE:], l1_eb0, l1_eW1, l1_eb1)
    (sums1,) = _sc_scatter(e2.reshape(-1), col)
    return _node(x1, sums1, l1_nW0, l1_nb0, l1_nW1, l1_nb1)


# speedup vs baseline: 2.9884x; 1.0076x over previous
import functools

import jax
import jax.numpy as jnp
from jax import lax
from jax.experimental import pallas as pl
from jax.experimental.pallas import tpu as pltpu
from jax.experimental.pallas import tpu_sc as plsc

N_NODES = 10000
N_EDGES = 320000
D_NODE = 128
D_EDGE = 16
HID = 128

CH = 128
NCH = N_EDGES // CH
NW = 32
NP = 10240
SROWS = NP // 16

_mesh = plsc.VectorSubcoreMesh(core_axis_name="c", subcore_axis_name="s")


@functools.partial(
    pl.kernel,
    out_type=[jax.ShapeDtypeStruct((N_EDGES, HID), jnp.float32),
              jax.ShapeDtypeStruct((N_EDGES, HID), jnp.float32)],
    mesh=_mesh,
    scratch_types=[
        pltpu.VMEM((3, 2, CH), jnp.int32),
        pltpu.VMEM((3, CH, HID), jnp.float32),
        pltpu.VMEM((3, CH, HID), jnp.float32),
        pltpu.SemaphoreType.DMA, pltpu.SemaphoreType.DMA,
        pltpu.SemaphoreType.DMA, pltpu.SemaphoreType.DMA,
        pltpu.SemaphoreType.DMA, pltpu.SemaphoreType.DMA,
        pltpu.SemaphoreType.DMA, pltpu.SemaphoreType.DMA,
        pltpu.SemaphoreType.DMA, pltpu.SemaphoreType.DMA,
        pltpu.SemaphoreType.DMA, pltpu.SemaphoreType.DMA,
    ],
)
def _sc_gather(pr, pc, eidx, gr, gc, idx2, bufr, bufc,
               semr0, semr1, semr2, semc0, semc1, semc2,
               swr0, swr1, swr2, swc0, swc1, swc2):
    c = lax.axis_index("c")
    s = lax.axis_index("s")
    wid = s * 2 + c
    semr = (semr0, semr1, semr2)
    semc = (semc0, semc1, semc2)
    swr = (swr0, swr1, swr2)
    swc = (swc0, swc1, swc2)
    npair = ((NCH + NW - 1) // NW + 2) // 3

    def pair(jj, carry):
        for b in range(3):
            ch = wid + (jj * 3 + b) * NW

            @pl.when(ch < NCH)
            def _(b=b, ch=ch):
                base = ch * CH
                pltpu.sync_copy(eidx.at[:, pl.ds(base, CH)], idx2.at[b])
                pltpu.async_copy(pr.at[idx2.at[b, 0]], bufr.at[b], semr[b])
                pltpu.async_copy(pc.at[idx2.at[b, 1]], bufc.at[b], semc[b])

        for b in range(3):
            ch = wid + (jj * 3 + b) * NW

            @pl.when(ch < NCH)
            def _(b=b, ch=ch):
                base = ch * CH
                pltpu.make_async_copy(pr.at[idx2.at[b, 0]], bufr.at[b],
                                      semr[b]).wait()
                pltpu.async_copy(bufr.at[b], gr.at[pl.ds(base, CH)], swr[b])
                pltpu.make_async_copy(pc.at[idx2.at[b, 1]], bufc.at[b],
                                      semc[b]).wait()
                pltpu.async_copy(bufc.at[b], gc.at[pl.ds(base, CH)], swc[b])

        for b in range(3):
            ch = wid + (jj * 3 + b) * NW

            @pl.when(ch < NCH)
            def _(b=b, ch=ch):
                base = ch * CH
                pltpu.make_async_copy(bufr.at[b], gr.at[pl.ds(base, CH)],
                                      swr[b]).wait()
                pltpu.make_async_copy(bufc.at[b], gc.at[pl.ds(base, CH)],
                                      swc[b]).wait()

        return carry

    lax.fori_loop(0, npair, pair, 0)


_SCATTER_SCRATCH = [
    pltpu.VMEM((2, CH), jnp.int32),
    pltpu.VMEM((2, CH * D_EDGE), jnp.float32),
    pltpu.VMEM((2, CH, 128), jnp.float32),
    pltpu.VMEM_SHARED((NP, 128), jnp.float32),
    pltpu.SemaphoreType.DMA,
    pltpu.SemaphoreType.DMA,
]

_SCATTER_OUT = [jax.ShapeDtypeStruct((2 * NP, 128), jnp.float32)]


@functools.partial(pl.kernel, out_type=_SCATTER_OUT, mesh=_mesh,
                   scratch_types=_SCATTER_SCRATCH)
def _sc_scatter(ev, coli, sums, idx, vals, stage, accs, sa0, sa1):
    c = lax.axis_index("c")
    s = lax.axis_index("s")
    wid = s * 2 + c
    sads = (sa0, sa1)

    def zb(i, carry):
        for b in range(2):
            for q in range(8):
                stage[b, i, pl.ds(q * 16, 16)] = jnp.zeros((16,), jnp.float32)
        return carry

    lax.fori_loop(0, CH, zb, 0)

    def zs(k, carry):
        pltpu.sync_copy(stage.at[0], accs.at[pl.ds(s * SROWS + k * CH, CH)])
        return carry

    lax.fori_loop(0, SROWS // CH, zs, 0)

    def ob(i, carry):
        for b in range(2):
            stage[b, i, pl.ds(16, 16)] = jnp.full((16,), 1.0, jnp.float32)
        return carry

    lax.fori_loop(0, CH, ob, 0)
    plsc.subcore_barrier()

    npair = ((NCH + NW - 1) // NW + 1) // 2

    def pair(jj, carry):
        for b in range(2):
            ch = wid + (jj * 2 + b) * NW

            @pl.when(ch < NCH)
            def _(b=b, ch=ch):
                base = ch * CH
                pltpu.sync_copy(coli.at[pl.ds(base, CH)], idx.at[b])
                pltpu.sync_copy(ev.at[pl.ds(base * D_EDGE, CH * D_EDGE)],
                                vals.at[b])

                def exp(i, carry2):
                    stage[b, i, pl.ds(0, 16)] = vals[b, pl.ds(i * D_EDGE, 16)]
                    return carry2

                lax.fori_loop(0, CH, exp, 0)
                pltpu.async_copy(stage.at[b], accs.at[idx.at[b]], sads[b],
                                 add=True)

        for b in range(2):
            ch = wid + (jj * 2 + b) * NW

            @pl.when(ch < NCH)
            def _(b=b, ch=ch):
                pltpu.make_async_copy(stage.at[b], accs.at[idx.at[b]],
                                      sads[b]).wait()

        return carry

    lax.fori_loop(0, npair, pair, 0)
    plsc.subcore_barrier()

    def out(k, carry):
        pltpu.sync_copy(accs.at[pl.ds(s * SROWS + k * CH, CH)], stage.at[0])
        pltpu.sync_copy(stage.at[0],
                        sums.at[pl.ds(c * NP + s * SROWS + k * CH, CH)])
        return carry

    lax.fori_loop(0, SROWS // CH, out, 0)


def _proj_body(x_ref, wa_ref, wb_ref, pr_ref, pc_ref):
    xb = x_ref[...]
    pr_ref[...] = jnp.dot(xb, wa_ref[...], preferred_element_type=jnp.float32)
    pc_ref[...] = jnp.dot(xb, wb_ref[...], preferred_element_type=jnp.float32)


def _proj(x, wa, wb):
    B = 2000
    return pl.pallas_call(
        _proj_body,
        grid=(N_NODES // B,),
        in_specs=[pl.BlockSpec((B, D_NODE), lambda i: (i, 0)),
                  pl.BlockSpec((D_NODE, HID), lambda i: (0, 0)),
                  pl.BlockSpec((D_NODE, HID), lambda i: (0, 0))],
        out_specs=[pl.BlockSpec((B, HID), lambda i: (i, 0)),
                   pl.BlockSpec((B, HID), lambda i: (i, 0))],
        out_shape=[jax.ShapeDtypeStruct((N_NODES, HID), jnp.float32),
                   jax.ShapeDtypeStruct((N_NODES, HID), jnp.float32)],
    )(x, wa, wb)


def _combine_body(gr_ref, gc_ref, ea_ref, wc_ref, b0_ref, w1_ref, b1_ref, out_ref):
    h = (gr_ref[...] + gc_ref[...]
         + jnp.dot(ea_ref[...], wc_ref[...], preferred_element_type=jnp.float32)
         + b0_ref[...])
    h = jnp.maximum(h, 0.0)
    out_ref[...] = jnp.dot(h, w1_ref[...], preferred_element_type=jnp.float32) + b1_ref[...]


def _combine(gr, gc, ea, wc, b0, w1, b1):
    B = 2000
    return pl.pallas_call(
        _combine_body,
        grid=(N_EDGES // B,),
        in_specs=[pl.BlockSpec((B, HID), lambda i: (i, 0)),
                  pl.BlockSpec((B, HID), lambda i: (i, 0)),
                  pl.BlockSpec((B, D_EDGE), lambda i: (i, 0)),
                  pl.BlockSpec((D_EDGE, HID), lambda i: (0, 0)),
                  pl.BlockSpec((1, HID), lambda i: (0, 0)),
                  pl.BlockSpec((HID, D_EDGE), lambda i: (0, 0)),
                  pl.BlockSpec((1, D_EDGE), lambda i: (0, 0))],
        out_specs=pl.BlockSpec((B, D_EDGE), lambda i: (i, 0)),
        out_shape=jax.ShapeDtypeStruct((N_EDGES, D_EDGE), jnp.float32),
    )(gr, gc, ea, wc, b0.reshape(1, HID), w1, b1.reshape(1, D_EDGE))


def _node_body(x_ref, s0_ref, s1_ref, c0_ref, c1_ref, wa_ref, wb_ref, b0_ref,
               w1_ref, b1_ref, out_ref):
    cnt = jnp.maximum(c0_ref[...] + c1_ref[...], 1.0)
    agg = (s0_ref[...] + s1_ref[...]) / cnt
    h = (jnp.dot(x_ref[...], wa_ref[...], preferred_element_type=jnp.float32)
         + jnp.dot(agg, wb_ref[...], preferred_element_type=jnp.float32)
         + b0_ref[...])
    h = jnp.maximum(h, 0.0)
    out_ref[...] = jnp.dot(h, w1_ref[...], preferred_element_type=jnp.float32) + b1_ref[...]


def _node_proj_body(x_ref, s0_ref, s1_ref, c0_ref, c1_ref, wa_ref, wb_ref,
                    b0_ref, w1_ref, b1_ref, wea_ref, web_ref,
                    out_ref, pr_ref, pc_ref):
    cnt = jnp.maximum(c0_ref[...] + c1_ref[...], 1.0)
    agg = (s0_ref[...] + s1_ref[...]) / cnt
    h = (jnp.dot(x_ref[...], wa_ref[...], preferred_element_type=jnp.float32)
         + jnp.dot(agg, wb_ref[...], preferred_element_type=jnp.float32)
         + b0_ref[...])
    h = jnp.maximum(h, 0.0)
    xn = jnp.dot(h, w1_ref[...], preferred_element_type=jnp.float32) + b1_ref[...]
    out_ref[...] = xn
    pr_ref[...] = jnp.dot(xn, wea_ref[...], preferred_element_type=jnp.float32)
    pc_ref[...] = jnp.dot(xn, web_ref[...], preferred_element_type=jnp.float32)


def _node_slices(sums):
    s0 = sums[:N_NODES, :D_EDGE]
    s1 = sums[NP:NP + N_NODES, :D_EDGE]
    c0 = sums[:N_NODES, D_EDGE:2 * D_EDGE]
    c1 = sums[NP:NP + N_NODES, D_EDGE:2 * D_EDGE]
    return s0, s1, c0, c1


def _node(x, sums, w0, b0, w1, b1):
    B = 2000
    nb = N_NODES // B
    s0, s1, c0, c1 = _node_slices(sums)
    return pl.pallas_call(
        _node_body,
        grid=(nb,),
        in_specs=[pl.BlockSpec((B, D_NODE), lambda i: (i, 0)),
                  pl.BlockSpec((B, D_EDGE), lambda i: (i, 0)),
                  pl.BlockSpec((B, D_EDGE), lambda i: (i, 0)),
                  pl.BlockSpec((B, D_EDGE), lambda i: (i, 0)),
                  pl.BlockSpec((B, D_EDGE), lambda i: (i, 0)),
                  pl.BlockSpec((D_NODE, HID), lambda i: (0, 0)),
                  pl.BlockSpec((D_EDGE, HID), lambda i: (0, 0)),
                  pl.BlockSpec((1, HID), lambda i: (0, 0)),
                  pl.BlockSpec((HID, D_NODE), lambda i: (0, 0)),
                  pl.BlockSpec((1, D_NODE), lambda i: (0, 0))],
        out_specs=pl.BlockSpec((B, D_NODE), lambda i: (i, 0)),
        out_shape=jax.ShapeDtypeStruct((N_NODES, D_NODE), jnp.float32),
    )(x, s0, s1, c0, c1, w0[:D_NODE], w0[D_NODE:], b0.reshape(1, HID), w1,
      b1.reshape(1, D_NODE))


def _node_proj(x, sums, w0, b0, w1, b1, wea, web):
    B = 2000
    nb = N_NODES // B
    s0, s1, c0, c1 = _node_slices(sums)
    return pl.pallas_call(
        _node_proj_body,
        grid=(nb,),
        in_specs=[pl.BlockSpec((B, D_NODE), lambda i: (i, 0)),
                  pl.BlockSpec((B, D_EDGE), lambda i: (i, 0)),
                  pl.BlockSpec((B, D_EDGE), lambda i: (i, 0)),
                  pl.BlockSpec((B, D_EDGE), lambda i: (i, 0)),
                  pl.BlockSpec((B, D_EDGE), lambda i: (i, 0)),
                  pl.BlockSpec((D_NODE, HID), lambda i: (0, 0)),
                  pl.BlockSpec((D_EDGE, HID), lambda i: (0, 0)),
                  pl.BlockSpec((1, HID), lambda i: (0, 0)),
                  pl.BlockSpec((HID, D_NODE), lambda i: (0, 0)),
                  pl.BlockSpec((1, D_NODE), lambda i: (0, 0)),
                  pl.BlockSpec((D_NODE, HID), lambda i: (0, 0)),
                  pl.BlockSpec((D_NODE, HID), lambda i: (0, 0))],
        out_specs=[pl.BlockSpec((B, D_NODE), lambda i: (i, 0)),
                   pl.BlockSpec((B, HID), lambda i: (i, 0)),
                   pl.BlockSpec((B, HID), lambda i: (i, 0))],
        out_shape=[jax.ShapeDtypeStruct((N_NODES, D_NODE), jnp.float32),
                   jax.ShapeDtypeStruct((N_NODES, HID), jnp.float32),
                   jax.ShapeDtypeStruct((N_NODES, HID), jnp.float32)],
    )(x, s0, s1, c0, c1, w0[:D_NODE], w0[D_NODE:], b0.reshape(1, HID), w1,
      b1.reshape(1, D_NODE), wea, web)


def kernel(x, edge_attr, l0_eW0, l0_eb0, l0_eW1, l0_eb1, l0_nW0, l0_nb0,
           l0_nW1, l0_nb1, l1_eW0, l1_eb0, l1_eW1, l1_eb1, l1_nW0, l1_nb0,
           l1_nW1, l1_nb1, edge_index):
    col = edge_index[1]
    pr, pc = _proj(x, l0_eW0[:D_NODE], l0_eW0[D_NODE:2 * D_NODE])
    gr, gc = _sc_gather(pr, pc, edge_index)
    e1 = _combine(gr, gc, edge_attr, l0_eW0[2 * D_NODE:], l0_eb0, l0_eW1,
                  l0_eb1)
    (sums0,) = _sc_scatter(e1.reshape(-1), col)
    x1, pr1, pc1 = _node_proj(x, sums0, l0_nW0, l0_nb0, l0_nW1, l0_nb1,
                              l1_eW0[:D_NODE], l1_eW0[D_NODE:2 * D_NODE])
    gr1, gc1 = _sc_gather(pr1, pc1, edge_index)
    e2 = _combine(gr1, gc1, e1, l1_eW0[2 * D_NODE:], l1_eb0, l1_eW1, l1_eb1)
    (sums1,) = _sc_scatter(e2.reshape(-1), col)
    return _node(x1, sums1, l1_nW0, l1_nb0, l1_nW1, l1_nb1)
